# baseline (speedup 1.0000x reference)
import functools

import jax
import jax.numpy as jnp
from jax.experimental import pallas as pl

_EPS = 1e-5
_NQ = 16
_KS = [0.5, 0.5]
_LN = 2
_F32 = jnp.float32


def _lrelu(x):
    return jnp.where(x >= 0, x, 0.01 * x)


def _iota(shape, dim):
    return jax.lax.broadcasted_iota(jnp.int32, shape, dim)


def _stats_update(s_ref, h):
    b = pl.program_id(0)
    s = jnp.concatenate([jnp.sum(h, axis=0, keepdims=True),
                         jnp.sum(h * h, axis=0, keepdims=True)], axis=0)

    @pl.when(b == 0)
    def _():
        s_ref[...] = s

    @pl.when(b > 0)
    def _():
        s_ref[...] = s_ref[...] + s


def _k_l0(x_ref, w_ref, h_ref, s_ref):
    x = x_ref[0]
    n = x.shape[0]
    p = n * n
    ri = ((_iota((p, n), 0) // n) == _iota((p, n), 1)).astype(_F32)
    rj = ((_iota((p, n), 0) % n) == _iota((p, n), 1)).astype(_F32)
    xi = jnp.dot(ri, x, preferred_element_type=_F32)
    xj = jnp.dot(rj, x, preferred_element_type=_F32)
    d = jnp.abs(xi - xj)
    h = jnp.dot(d, w_ref[...], preferred_element_type=_F32)
    h_ref[0] = h
    _stats_update(s_ref, h)


def _k_mid(h_ref, a_ref, c_ref, w_ref, o_ref, s_ref):
    t = _lrelu(h_ref[0] * a_ref[...] + c_ref[...])
    h = jnp.dot(t, w_ref[...], preferred_element_type=_F32)
    o_ref[0] = h
    _stats_update(s_ref, h)


def _k_last(h_ref, a_ref, c_ref, w_ref, b_ref, o_ref, *, n):
    t = _lrelu(h_ref[0] * a_ref[...] + c_ref[...])
    l = jnp.dot(t, w_ref[...], preferred_element_type=_F32) + b_ref[0, 0]
    p = n * n
    ri = ((_iota((p, n), 0) // n) == _iota((p, n), 1)).astype(_F32)
    m = jnp.max(l)
    e = jnp.exp(l - m)
    den = jax.lax.dot_general(ri, e, (((0,), (0,)), ((), ())),
                              preferred_element_type=_F32)
    o_ref[0] = e * jnp.dot(ri, 1.0 / den, preferred_element_type=_F32)


def _k_gcn(an_ref, ao_ref, x_ref, w_ref, b_ref, o_ref, *, logsm):
    x = x_ref[0]
    d = x.shape[-1]
    x1 = jnp.dot(an_ref[0], x, preferred_element_type=_F32)
    x2 = jnp.dot(ao_ref[0], x, preferred_element_type=_F32)
    w = w_ref[...]
    o = (jnp.dot(x1, w[:d], preferred_element_type=_F32)
         + jnp.dot(x2, w[d:], preferred_element_type=_F32) + b_ref[...])
    if logsm:
        mx = jnp.max(o, axis=-1, keepdims=True)
        o = o - mx
        o = o - jnp.log(jnp.sum(jnp.exp(o), axis=-1, keepdims=True))
    o_ref[0] = o


def _k_pool(x_ref, a_ref, w_ref, b_ref, na_ref, nx_ref, p_ref, *, ns, kint, newn):
    x = x_ref[0]
    adj = a_ref[0]
    n = x.shape[0]
    raw = (jnp.sum(x * w_ref[...], axis=-1, keepdims=True) + b_ref[0, 0]) / 100.0
    sc = jax.nn.sigmoid(raw)
    sr = jnp.transpose(sc)
    rowi = _iota((n, n), 0)
    colj = _iota((n, n), 1)
    m2 = ((sc > sr) | ((sc == sr) & (rowi < colj))) & (rowi < ns)
    rank = jnp.sum(m2.astype(_F32), axis=0, keepdims=True)
    cj = _iota((1, n), 1)
    rank = jnp.where(rank < kint, rank, 1e9)
    rankf = jnp.where(cj < ns, rank, (kint + cj - ns).astype(_F32))
    pm = (_iota((newn, n), 0).astype(_F32) == rankf).astype(_F32)
    vals = jnp.dot(pm, sc, preferred_element_type=_F32)
    nx_ref[0] = jnp.dot(pm, x, preferred_element_type=_F32) * vals
    ta = jnp.dot(pm, adj, preferred_element_type=_F32)
    na_ref[0] = jax.lax.dot_general(ta, pm, (((1,), (1,)), ((), ())),
                                    preferred_element_type=_F32)
    p_ref[0] = pm


def _k_unpool(p_ref, x_ref, d_ref, o_ref):
    o_ref[0] = jax.lax.dot_general(p_ref[0], x_ref[0],
                                   (((0,), (0,)), ((), ())),
                                   preferred_element_type=_F32) + d_ref[0]


def _full(shape):
    nd = len(shape)
    return pl.BlockSpec(shape, lambda b: (0,) * nd)


def _perb(shape):
    nd = len(shape)
    return pl.BlockSpec((1,) + shape, lambda b: (b,) + (0,) * nd)


def _pallas_mlp(p, x):
    b, n, d = x.shape
    pp = n * n
    ntot = b * pp
    chs = [d, 192, 192, 96, 96]
    h, s = pl.pallas_call(
        _k_l0,
        grid=(b,),
        in_specs=[_perb((n, d)), _full((d, 192))],
        out_specs=[_perb((pp, 192)), _full((2, 192))],
        out_shape=[jax.ShapeDtypeStruct((b, pp, 192), _F32),
                   jax.ShapeDtypeStruct((2, 192), _F32)],
    )(x, p['w0'])
    for i in range(1, 4):
        cin, cout = chs[i], chs[i + 1]
        mean = s[0] / ntot
        var = s[1] / ntot - mean * mean
        scl = p['g%d' % (i - 1)] / jnp.sqrt(var + _EPS)
        sft = p['be%d' % (i - 1)] - mean * scl
        h, s = pl.pallas_call(
            _k_mid,
            grid=(b,),
            in_specs=[_perb((pp, cin)), _full((1, cin)), _full((1, cin)),
                      _full((cin, cout))],
            out_specs=[_perb((pp, cout)), _full((2, cout))],
            out_shape=[jax.ShapeDtypeStruct((b, pp, cout), _F32),
                       jax.ShapeDtypeStruct((2, cout), _F32)],
        )(h, scl[None], sft[None], p['w%d' % i])
    mean = s[0] / ntot
    var = s[1] / ntot - mean * mean
    scl = p['g3'] / jnp.sqrt(var + _EPS)
    sft = p['be3'] - mean * scl
    a_col = pl.pallas_call(
        functools.partial(_k_last, n=n),
        grid=(b,),
        in_specs=[_perb((pp, 96)), _full((1, 96)), _full((1, 96)),
                  _full((96, 1)), _full((1, 1))],
        out_specs=_perb((pp, 1)),
        out_shape=jax.ShapeDtypeStruct((b, pp, 1), _F32),
    )(h, scl[None], sft[None], p['w4'], p['b4'].reshape(1, 1))
    return a_col.reshape(b, n, n)


def _pallas_gcn(p, a_new, a_old, x, logsm=False):
    b, n, d = x.shape
    dout = p['w'].shape[1]
    return pl.pallas_call(
        functools.partial(_k_gcn, logsm=logsm),
        grid=(b,),
        in_specs=[_perb((n, n)), _perb((n, n)), _perb((n, d)),
                  _full((2 * d, dout)), _full((1, dout))],
        out_specs=_perb((n, dout)),
        out_shape=jax.ShapeDtypeStruct((b, n, dout), _F32),
    )(a_new, a_old, x, p['w'], p['b'][None])


def _pallas_pool(p, k, adj, x):
    b, n, d = x.shape
    ns = n - _NQ
    kint = int(k * ns)
    newn = kint + _NQ
    na, nx, pm = pl.pallas_call(
        functools.partial(_k_pool, ns=ns, kint=kint, newn=newn),
        grid=(b,),
        in_specs=[_perb((n, d)), _perb((n, n)), _full((1, d)), _full((1, 1))],
        out_specs=[_perb((newn, newn)), _perb((newn, d)), _perb((newn, n))],
        out_shape=[jax.ShapeDtypeStruct((b, newn, newn), _F32),
                   jax.ShapeDtypeStruct((b, newn, d), _F32),
                   jax.ShapeDtypeStruct((b, newn, n), _F32)],
    )(x, adj, jnp.transpose(p['w']), p['b'].reshape(1, 1))
    return na, nx, pm


def _pallas_unpool_add(pm, xp, down):
    b, newn, n = pm.shape
    d = xp.shape[-1]
    return pl.pallas_call(
        _k_unpool,
        grid=(b,),
        in_specs=[_perb((newn, n)), _perb((newn, d)), _perb((n, d))],
        out_specs=_perb((n, d)),
        out_shape=jax.ShapeDtypeStruct((b, n, d), _F32),
    )(pm, xp, down)


def kernel(A_init, X, params):
    org_x = X
    a_old = A_init
    a_new = _pallas_mlp(params['start_mlp'], X)
    x = _pallas_gcn(params['start_gcn'], a_new, a_old, X)
    adj, downs, pms = [], [], []
    for i in range(_LN):
        a_old = a_new
        a_new = _pallas_mlp(params['down_mlp_%d' % i], x)
        x = _pallas_gcn(params['down_gcn_%d' % i], a_new, a_old, x)
        adj.append(a_new)
        downs.append(x)
        a_new, x, pm = _pallas_pool(params['pool_%d' % i], _KS[i], a_new, x)
        pms.append(pm)
    a_old = a_new
    a_new = _pallas_mlp(params['bottom_mlp'], x)
    x = _pallas_gcn(params['bottom_gcn'], a_new, a_old, x)
    for i in range(_LN):
        u = _LN - 1 - i
        a_old = adj[u]
        x = _pallas_unpool_add(pms[u], x, downs[u])
        a_new = _pallas_mlp(params['up_mlp_%d' % u], x)
        x = _pallas_gcn(params['up_gcn_%d' % u], a_new, a_old, x)
    x = jnp.concatenate([x, org_x], axis=-1)
    a_old = a_new
    a_new = _pallas_mlp(params['out_mlp'], x)
    return _pallas_gcn(params['out_gcn'], a_new, a_old, x, logsm=True)


# speedup vs baseline: 1.0373x; 1.0373x over previous
import functools

import jax
import jax.numpy as jnp
from jax.experimental import pallas as pl

_EPS = 1e-5
_NQ = 16
_KS = [0.5, 0.5]
_LN = 2
_F32 = jnp.float32


def _lrelu(x):
    return jnp.where(x >= 0, x, 0.01 * x)


def _iota(shape, dim):
    return jax.lax.broadcasted_iota(jnp.int32, shape, dim)


def _stats_update(s_ref, h):
    b = pl.program_id(0)
    ones = jnp.ones((1, h.shape[0]), _F32)
    s0 = jnp.dot(ones, h, preferred_element_type=_F32)
    s1 = jnp.dot(ones, h * h, preferred_element_type=_F32)
    s = jnp.concatenate([s0, s1], axis=0)

    @pl.when(b == 0)
    def _():
        s_ref[...] = s

    @pl.when(b > 0)
    def _():
        s_ref[...] = s_ref[...] + s


def _affine_lrelu(h, s_ref, g_ref, be_ref, ntot):
    mean = s_ref[0:1] / ntot
    var = s_ref[1:2] / ntot - mean * mean
    scl = g_ref[...] * jax.lax.rsqrt(var + _EPS)
    sft = be_ref[...] - mean * scl
    y = h * scl + sft
    return jnp.maximum(y, 0.01 * y)


def _k_l0(x_ref, w_ref, h_ref, s_ref):
    x = x_ref[0]
    n = x.shape[0]
    p = n * n
    ri = ((_iota((p, n), 0) // n) == _iota((p, n), 1)).astype(_F32)
    rj = ((_iota((p, n), 0) % n) == _iota((p, n), 1)).astype(_F32)
    xi = jnp.dot(ri, x, preferred_element_type=_F32)
    xj = jnp.dot(rj, x, preferred_element_type=_F32)
    d = jnp.abs(xi - xj)
    h = jnp.dot(d, w_ref[...], preferred_element_type=_F32)
    h_ref[0] = h
    _stats_update(s_ref, h)


def _k_mid(h_ref, s_ref, g_ref, be_ref, w_ref, o_ref, so_ref, *, ntot):
    t = _affine_lrelu(h_ref[0], s_ref, g_ref, be_ref, ntot)
    h = jnp.dot(t, w_ref[...], preferred_element_type=_F32)
    o_ref[0] = h
    _stats_update(so_ref, h)


def _k_last(h_ref, s_ref, g_ref, be_ref, w_ref, o_ref, *, n, ntot):
    t = _affine_lrelu(h_ref[0], s_ref, g_ref, be_ref, ntot)
    t3 = t.reshape(n, n, t.shape[1])
    l = jnp.sum(t3 * w_ref[...][None], axis=-1)
    e = jnp.exp(l)
    o_ref[0] = e / jnp.sum(e, axis=-1, keepdims=True)


def _k_gcn(an_ref, ao_ref, x_ref, w_ref, b_ref, o_ref, *, logsm):
    x = x_ref[0]
    d = x.shape[-1]
    x1 = jnp.dot(an_ref[0], x, preferred_element_type=_F32)
    x2 = jnp.dot(ao_ref[0], x, preferred_element_type=_F32)
    w = w_ref[...]
    o = (jnp.dot(x1, w[:d], preferred_element_type=_F32)
         + jnp.dot(x2, w[d:], preferred_element_type=_F32) + b_ref[...])
    if logsm:
        mx = jnp.max(o, axis=-1, keepdims=True)
        o = o - mx
        o = o - jnp.log(jnp.sum(jnp.exp(o), axis=-1, keepdims=True))
    o_ref[0] = o


def _k_pool(x_ref, a_ref, w_ref, b_ref, na_ref, nx_ref, p_ref, *, ns, kint, newn):
    x = x_ref[0]
    adj = a_ref[0]
    n = x.shape[0]
    raw = (jnp.sum(x * w_ref[...], axis=-1, keepdims=True) + b_ref[0, 0]) / 100.0
    sc = jax.nn.sigmoid(raw)
    sr = jnp.transpose(sc)
    rowi = _iota((n, n), 0)
    colj = _iota((n, n), 1)
    m2 = ((sc > sr) | ((sc == sr) & (rowi < colj))) & (rowi < ns)
    rank = jnp.sum(m2.astype(_F32), axis=0, keepdims=True)
    cj = _iota((1, n), 1)
    rank = jnp.where(rank < kint, rank, 1e9)
    rankf = jnp.where(cj < ns, rank, (kint + cj - ns).astype(_F32))
    pm = (_iota((newn, n), 0).astype(_F32) == rankf).astype(_F32)
    vals = jnp.dot(pm, sc, preferred_element_type=_F32)
    nx_ref[0] = jnp.dot(pm, x, preferred_element_type=_F32) * vals
    ta = jnp.dot(pm, adj, preferred_element_type=_F32)
    na_ref[0] = jax.lax.dot_general(ta, pm, (((1,), (1,)), ((), ())),
                                    preferred_element_type=_F32)
    p_ref[0] = pm


def _k_unpool(p_ref, x_ref, d_ref, o_ref):
    o_ref[0] = jax.lax.dot_general(p_ref[0], x_ref[0],
                                   (((0,), (0,)), ((), ())),
                                   preferred_element_type=_F32) + d_ref[0]


def _full(shape):
    nd = len(shape)
    return pl.BlockSpec(shape, lambda b: (0,) * nd)


def _perb(shape):
    nd = len(shape)
    return pl.BlockSpec((1,) + shape, lambda b: (b,) + (0,) * nd)


def _pallas_mlp(p, x):
    b, n, d = x.shape
    pp = n * n
    ntot = b * pp
    chs = [d, 192, 192, 96, 96]
    h, s = pl.pallas_call(
        _k_l0,
        grid=(b,),
        in_specs=[_perb((n, d)), _full((d, 192))],
        out_specs=[_perb((pp, 192)), _full((2, 192))],
        out_shape=[jax.ShapeDtypeStruct((b, pp, 192), _F32),
                   jax.ShapeDtypeStruct((2, 192), _F32)],
    )(x, p['w0'])
    for i in range(1, 4):
        cin, cout = chs[i], chs[i + 1]
        h, s = pl.pallas_call(
            functools.partial(_k_mid, ntot=float(ntot)),
            grid=(b,),
            in_specs=[_perb((pp, cin)), _full((2, cin)), _full((1, cin)),
                      _full((1, cin)), _full((cin, cout))],
            out_specs=[_perb((pp, cout)), _full((2, cout))],
            out_shape=[jax.ShapeDtypeStruct((b, pp, cout), _F32),
                       jax.ShapeDtypeStruct((2, cout), _F32)],
        )(h, s, p['g%d' % (i - 1)][None], p['be%d' % (i - 1)][None],
          p['w%d' % i])
    return pl.pallas_call(
        functools.partial(_k_last, n=n, ntot=float(ntot)),
        grid=(b,),
        in_specs=[_perb((pp, 96)), _full((2, 96)), _full((1, 96)),
                  _full((1, 96)), _full((1, 96))],
        out_specs=_perb((n, n)),
        out_shape=jax.ShapeDtypeStruct((b, n, n), _F32),
    )(h, s, p['g3'][None], p['be3'][None], jnp.transpose(p['w4']))


def _pallas_gcn(p, a_new, a_old, x, logsm=False):
    b, n, d = x.shape
    dout = p['w'].shape[1]
    return pl.pallas_call(
        functools.partial(_k_gcn, logsm=logsm),
        grid=(b,),
        in_specs=[_perb((n, n)), _perb((n, n)), _perb((n, d)),
                  _full((2 * d, dout)), _full((1, dout))],
        out_specs=_perb((n, dout)),
        out_shape=jax.ShapeDtypeStruct((b, n, dout), _F32),
    )(a_new, a_old, x, p['w'], p['b'][None])


def _pallas_pool(p, k, adj, x):
    b, n, d = x.shape
    ns = n - _NQ
    kint = int(k * ns)
    newn = kint + _NQ
    na, nx, pm = pl.pallas_call(
        functools.partial(_k_pool, ns=ns, kint=kint, newn=newn),
        grid=(b,),
        in_specs=[_perb((n, d)), _perb((n, n)), _full((1, d)), _full((1, 1))],
        out_specs=[_perb((newn, newn)), _perb((newn, d)), _perb((newn, n))],
        out_shape=[jax.ShapeDtypeStruct((b, newn, newn), _F32),
                   jax.ShapeDtypeStruct((b, newn, d), _F32),
                   jax.ShapeDtypeStruct((b, newn, n), _F32)],
    )(x, adj, jnp.transpose(p['w']), p['b'].reshape(1, 1))
    return na, nx, pm


def _pallas_unpool_add(pm, xp, down):
    b, newn, n = pm.shape
    d = xp.shape[-1]
    return pl.pallas_call(
        _k_unpool,
        grid=(b,),
        in_specs=[_perb((newn, n)), _perb((newn, d)), _perb((n, d))],
        out_specs=_perb((n, d)),
        out_shape=jax.ShapeDtypeStruct((b, n, d), _F32),
    )(pm, xp, down)


def kernel(A_init, X, params):
    org_x = X
    a_old = A_init
    a_new = _pallas_mlp(params['start_mlp'], X)
    x = _pallas_gcn(params['start_gcn'], a_new, a_old, X)
    adj, downs, pms = [], [], []
    for i in range(_LN):
        a_old = a_new
        a_new = _pallas_mlp(params['down_mlp_%d' % i], x)
        x = _pallas_gcn(params['down_gcn_%d' % i], a_new, a_old, x)
        adj.append(a_new)
        downs.append(x)
        a_new, x, pm = _pallas_pool(params['pool_%d' % i], _KS[i], a_new, x)
        pms.append(pm)
    a_old = a_new
    a_new = _pallas_mlp(params['bottom_mlp'], x)
    x = _pallas_gcn(params['bottom_gcn'], a_new, a_old, x)
    for i in range(_LN):
        u = _LN - 1 - i
        a_old = adj[u]
        x = _pallas_unpool_add(pms[u], x, downs[u])
        a_new = _pallas_mlp(params['up_mlp_%d' % u], x)
        x = _pallas_gcn(params['up_gcn_%d' % u], a_new, a_old, x)
    x = jnp.concatenate([x, org_x], axis=-1)
    a_old = a_new
    a_new = _pallas_mlp(params['out_mlp'], x)
    return _pallas_gcn(params['out_gcn'], a_new, a_old, x, logsm=True)


# speedup vs baseline: 1.5340x; 1.4789x over previous
import functools

import jax
import jax.numpy as jnp
import numpy as np
from jax.experimental import pallas as pl

_EPS = 1e-5
_NQ = 16
_KS = [0.5, 0.5]
_LN = 2
_F32 = jnp.float32


def _iota(shape, dim):
    return jax.lax.broadcasted_iota(jnp.int32, shape, dim)


@functools.lru_cache(maxsize=None)
def _tri_np(n):
    p2 = n * (n - 1) // 2
    p2p = -(-p2 // 8) * 8
    iu, ju = np.triu_indices(n, 1)
    r2 = np.zeros((p2p, n), np.float32)
    t2 = np.zeros((p2p, n), np.float32)
    r2[np.arange(p2), iu] = 1.0
    t2[np.arange(p2), ju] = 1.0
    return p2, p2p, r2, t2


def _rowmask(p2, p2p):
    return (_iota((p2p, 1), 0) < p2).astype(_F32)


def _stats_update(s_ref, h):
    b = pl.program_id(0)
    ones = jnp.ones((1, h.shape[0]), _F32)
    s0 = jnp.dot(ones, h, preferred_element_type=_F32)
    s1 = jnp.dot(ones, h * h, preferred_element_type=_F32)
    s = jnp.concatenate([s0, s1], axis=0)

    @pl.when(b == 0)
    def _():
        s_ref[...] = s

    @pl.when(b > 0)
    def _():
        s_ref[...] = s_ref[...] + s


def _scale_shift(s_ref, hd, g_ref, be_ref, ntot, bn):
    s0 = 2.0 * s_ref[0:1] + bn * hd
    s1 = 2.0 * s_ref[1:2] + bn * hd * hd
    mean = s0 / ntot
    var = s1 / ntot - mean * mean
    scl = g_ref[...] * jax.lax.rsqrt(var + _EPS)
    sft = be_ref[...] - mean * scl
    return scl, sft


def _act(y):
    return jnp.maximum(y, 0.01 * y)


def _k_mom(x_ref, r2_ref, t2_ref, m_ref, sd_ref):
    b = pl.program_id(0)
    x = x_ref[0]
    d = jnp.abs(jnp.dot(r2_ref[...], x, preferred_element_type=_F32)
                - jnp.dot(t2_ref[...], x, preferred_element_type=_F32))
    m = jax.lax.dot_general(d, d, (((0,), (0,)), ((), ())),
                            preferred_element_type=_F32)
    sd = jnp.dot(jnp.ones((1, d.shape[0]), _F32), d,
                 preferred_element_type=_F32)

    @pl.when(b == 0)
    def _():
        m_ref[...] = m
        sd_ref[...] = sd

    @pl.when(b > 0)
    def _():
        m_ref[...] = m_ref[...] + m
        sd_ref[...] = sd_ref[...] + sd


def _k_l01(x_ref, r2_ref, t2_ref, a_ref, c_ref, w0_ref, w1_ref,
           h_ref, s_ref, hd_ref, *, p2):
    x = x_ref[0]
    d = jnp.abs(jnp.dot(r2_ref[...], x, preferred_element_type=_F32)
                - jnp.dot(t2_ref[...], x, preferred_element_type=_F32))
    h0 = jnp.dot(d, w0_ref[...], preferred_element_type=_F32)
    t = _act(h0 * a_ref[...] + c_ref[...]) * _rowmask(p2, d.shape[0])
    h1 = jnp.dot(t, w1_ref[...], preferred_element_type=_F32)
    h_ref[0] = h1
    _stats_update(s_ref, h1)
    td = _act(c_ref[...])
    hd_ref[...] = jnp.dot(td, w1_ref[...], preferred_element_type=_F32)


def _k_mid2(h_ref, s_ref, hd_ref, g_ref, be_ref, w_ref,
            o_ref, so_ref, hdo_ref, *, ntot, bn, p2):
    scl, sft = _scale_shift(s_ref, hd_ref[...], g_ref, be_ref, ntot, bn)
    t = _act(h_ref[0] * scl + sft) * _rowmask(p2, h_ref.shape[1])
    h = jnp.dot(t, w_ref[...], preferred_element_type=_F32)
    o_ref[0] = h
    _stats_update(so_ref, h)
    td = _act(hd_ref[...] * scl + sft)
    hdo_ref[...] = jnp.dot(td, w_ref[...], preferred_element_type=_F32)


def _k_last2(h_ref, s_ref, hd_ref, g_ref, be_ref, w_ref, r2_ref, t2_ref,
             o_ref, *, n, ntot, bn, p2):
    scl, sft = _scale_shift(s_ref, hd_ref[...], g_ref, be_ref, ntot, bn)
    t = _act(h_ref[0] * scl + sft) * _rowmask(p2, h_ref.shape[1])
    l = jnp.sum(t * w_ref[...], axis=-1, keepdims=True)
    l1 = jax.lax.dot_general(r2_ref[...], l * t2_ref[...],
                             (((0,), (0,)), ((), ())),
                             preferred_element_type=_F32)
    l2 = jax.lax.dot_general(t2_ref[...], l * r2_ref[...],
                             (((0,), (0,)), ((), ())),
                             preferred_element_type=_F32)
    td = _act(hd_ref[...] * scl + sft)
    ld = jnp.sum(td * w_ref[...])
    eye = (_iota((n, n), 0) == _iota((n, n), 1)).astype(_F32)
    lg = l1 + l2 + ld * eye
    e = jnp.exp(lg)
    o_ref[0] = e / jnp.sum(e, axis=-1, keepdims=True)


def _k_gcn(an_ref, ao_ref, x_ref, w_ref, b_ref, o_ref, *, logsm):
    x = x_ref[0]
    d = x.shape[-1]
    x1 = jnp.dot(an_ref[0], x, preferred_element_type=_F32)
    x2 = jnp.dot(ao_ref[0], x, preferred_element_type=_F32)
    w = w_ref[...]
    o = (jnp.dot(x1, w[:d], preferred_element_type=_F32)
         + jnp.dot(x2, w[d:], preferred_element_type=_F32) + b_ref[...])
    if logsm:
        mx = jnp.max(o, axis=-1, keepdims=True)
        o = o - mx
        o = o - jnp.log(jnp.sum(jnp.exp(o), axis=-1, keepdims=True))
    o_ref[0] = o


def _k_pool(x_ref, a_ref, w_ref, b_ref, na_ref, nx_ref, p_ref, *, ns, kint, newn):
    x = x_ref[0]
    adj = a_ref[0]
    n = x.shape[0]
    raw = (jnp.sum(x * w_ref[...], axis=-1, keepdims=True) + b_ref[0, 0]) / 100.0
    sc = jax.nn.sigmoid(raw)
    sr = jnp.transpose(sc)
    rowi = _iota((n, n), 0)
    colj = _iota((n, n), 1)
    m2 = ((sc > sr) | ((sc == sr) & (rowi < colj))) & (rowi < ns)
    rank = jnp.sum(m2.astype(_F32), axis=0, keepdims=True)
    cj = _iota((1, n), 1)
    rank = jnp.where(rank < kint, rank, 1e9)
    rankf = jnp.where(cj < ns, rank, (kint + cj - ns).astype(_F32))
    pm = (_iota((newn, n), 0).astype(_F32) == rankf).astype(_F32)
    vals = jnp.dot(pm, sc, preferred_element_type=_F32)
    nx_ref[0] = jnp.dot(pm, x, preferred_element_type=_F32) * vals
    ta = jnp.dot(pm, adj, preferred_element_type=_F32)
    na_ref[0] = jax.lax.dot_general(ta, pm, (((1,), (1,)), ((), ())),
                                    preferred_element_type=_F32)
    p_ref[0] = pm


def _k_unpool(p_ref, x_ref, d_ref, o_ref):
    o_ref[0] = jax.lax.dot_general(p_ref[0], x_ref[0],
                                   (((0,), (0,)), ((), ())),
                                   preferred_element_type=_F32) + d_ref[0]


def _full(shape):
    nd = len(shape)
    return pl.BlockSpec(shape, lambda b: (0,) * nd)


def _perb(shape):
    nd = len(shape)
    return pl.BlockSpec((1,) + shape, lambda b: (b,) + (0,) * nd)


def _pallas_mlp(p, x):
    b, n, d = x.shape
    ntot = float(b * n * n)
    bn = float(b * n)
    p2, p2p, r2np, t2np = _tri_np(n)
    r2 = jnp.asarray(r2np)
    t2 = jnp.asarray(t2np)
    mom, sd = pl.pallas_call(
        _k_mom,
        grid=(b,),
        in_specs=[_perb((n, d)), _full((p2p, n)), _full((p2p, n))],
        out_specs=[_full((d, d)), _full((1, d))],
        out_shape=[jax.ShapeDtypeStruct((d, d), _F32),
                   jax.ShapeDtypeStruct((1, d), _F32)],
    )(x, r2, t2)
    mean0 = (2.0 * sd / ntot) @ p['w0']
    ex2 = jnp.sum(p['w0'] * ((2.0 * mom / ntot) @ p['w0']), axis=0)[None]
    var0 = ex2 - mean0 * mean0
    scl0 = p['g0'][None] * jax.lax.rsqrt(var0 + _EPS)
    sft0 = p['be0'][None] - mean0 * scl0
    h, s, hd = pl.pallas_call(
        functools.partial(_k_l01, p2=p2),
        grid=(b,),
        in_specs=[_perb((n, d)), _full((p2p, n)), _full((p2p, n)),
                  _full((1, 192)), _full((1, 192)),
                  _full((d, 192)), _full((192, 192))],
        out_specs=[_perb((p2p, 192)), _full((2, 192)), _full((1, 192))],
        out_shape=[jax.ShapeDtypeStruct((b, p2p, 192), _F32),
                   jax.ShapeDtypeStruct((2, 192), _F32),
                   jax.ShapeDtypeStruct((1, 192), _F32)],
    )(x, r2, t2, scl0, sft0, p['w0'], p['w1'])
    chs = [d, 192, 192, 96, 96]
    for i in range(2, 4):
        cin, cout = chs[i], chs[i + 1]
        h, s, hd = pl.pallas_call(
            functools.partial(_k_mid2, ntot=ntot, bn=bn, p2=p2),
            grid=(b,),
            in_specs=[_perb((p2p, cin)), _full((2, cin)), _full((1, cin)),
                      _full((1, cin)), _full((1, cin)), _full((cin, cout))],
            out_specs=[_perb((p2p, cout)), _full((2, cout)), _full((1, cout))],
            out_shape=[jax.ShapeDtypeStruct((b, p2p, cout), _F32),
                       jax.ShapeDtypeStruct((2, cout), _F32),
                       jax.ShapeDtypeStruct((1, cout), _F32)],
        )(h, s, hd, p['g%d' % (i - 1)][None], p['be%d' % (i - 1)][None],
          p['w%d' % i])
    return pl.pallas_call(
        functools.partial(_k_last2, n=n, ntot=ntot, bn=bn, p2=p2),
        grid=(b,),
        in_specs=[_perb((p2p, 96)), _full((2, 96)), _full((1, 96)),
                  _full((1, 96)), _full((1, 96)), _full((1, 96)),
                  _full((p2p, n)), _full((p2p, n))],
        out_specs=_perb((n, n)),
        out_shape=jax.ShapeDtypeStruct((b, n, n), _F32),
    )(h, s, hd, p['g3'][None], p['be3'][None], jnp.transpose(p['w4']),
      r2, t2)


def _pallas_gcn(p, a_new, a_old, x, logsm=False):
    b, n, d = x.shape
    dout = p['w'].shape[1]
    return pl.pallas_call(
        functools.partial(_k_gcn, logsm=logsm),
        grid=(b,),
        in_specs=[_perb((n, n)), _perb((n, n)), _perb((n, d)),
                  _full((2 * d, dout)), _full((1, dout))],
        out_specs=_perb((n, dout)),
        out_shape=jax.ShapeDtypeStruct((b, n, dout), _F32),
    )(a_new, a_old, x, p['w'], p['b'][None])


def _pallas_pool(p, k, adj, x):
    b, n, d = x.shape
    ns = n - _NQ
    kint = int(k * ns)
    newn = kint + _NQ
    na, nx, pm = pl.pallas_call(
        functools.partial(_k_pool, ns=ns, kint=kint, newn=newn),
        grid=(b,),
        in_specs=[_perb((n, d)), _perb((n, n)), _full((1, d)), _full((1, 1))],
        out_specs=[_perb((newn, newn)), _perb((newn, d)), _perb((newn, n))],
        out_shape=[jax.ShapeDtypeStruct((b, newn, newn), _F32),
                   jax.ShapeDtypeStruct((b, newn, d), _F32),
                   jax.ShapeDtypeStruct((b, newn, n), _F32)],
    )(x, adj, jnp.transpose(p['w']), p['b'].reshape(1, 1))
    return na, nx, pm


def _pallas_unpool_add(pm, xp, down):
    b, newn, n = pm.shape
    d = xp.shape[-1]
    return pl.pallas_call(
        _k_unpool,
        grid=(b,),
        in_specs=[_perb((newn, n)), _perb((newn, d)), _perb((n, d))],
        out_specs=_perb((n, d)),
        out_shape=jax.ShapeDtypeStruct((b, n, d), _F32),
    )(pm, xp, down)


def kernel(A_init, X, params):
    org_x = X
    a_old = A_init
    a_new = _pallas_mlp(params['start_mlp'], X)
    x = _pallas_gcn(params['start_gcn'], a_new, a_old, X)
    adj, downs, pms = [], [], []
    for i in range(_LN):
        a_old = a_new
        a_new = _pallas_mlp(params['down_mlp_%d' % i], x)
        x = _pallas_gcn(params['down_gcn_%d' % i], a_new, a_old, x)
        adj.append(a_new)
        downs.append(x)
        a_new, x, pm = _pallas_pool(params['pool_%d' % i], _KS[i], a_new, x)
        pms.append(pm)
    a_old = a_new
    a_new = _pallas_mlp(params['bottom_mlp'], x)
    x = _pallas_gcn(params['bottom_gcn'], a_new, a_old, x)
    for i in range(_LN):
        u = _LN - 1 - i
        a_old = adj[u]
        x = _pallas_unpool_add(pms[u], x, downs[u])
        a_new = _pallas_mlp(params['up_mlp_%d' % u], x)
        x = _pallas_gcn(params['up_gcn_%d' % u], a_new, a_old, x)
    x = jnp.concatenate([x, org_x], axis=-1)
    a_old = a_new
    a_new = _pallas_mlp(params['out_mlp'], x)
    return _pallas_gcn(params['out_gcn'], a_new, a_old, x, logsm=True)


# speedup vs baseline: 1.6042x; 1.0458x over previous
import functools

import jax
import jax.numpy as jnp
import numpy as np
from jax.experimental import pallas as pl

_EPS = 1e-5
_NQ = 16
_KS = [0.5, 0.5]
_LN = 2
_F32 = jnp.float32


def _iota(shape, dim):
    return jax.lax.broadcasted_iota(jnp.int32, shape, dim)


@functools.lru_cache(maxsize=None)
def _tri_np(n):
    p2 = n * (n - 1) // 2
    p2p = -(-p2 // 8) * 8
    iu, ju = np.triu_indices(n, 1)
    r2 = np.zeros((p2p, n), np.float32)
    t2 = np.zeros((p2p, n), np.float32)
    r2[np.arange(p2), iu] = 1.0
    t2[np.arange(p2), ju] = 1.0
    return p2, p2p, r2, t2


def _rowmask(p2, p2p):
    return (_iota((p2p, 1), 0) < p2).astype(_F32)


def _stats_update(s_ref, h):
    b = pl.program_id(0)
    ones = jnp.ones((1, h.shape[0]), _F32)
    s0 = jnp.dot(ones, h, preferred_element_type=_F32)
    s1 = jnp.dot(ones, h * h, preferred_element_type=_F32)
    s = jnp.concatenate([s0, s1], axis=0)

    @pl.when(b == 0)
    def _():
        s_ref[...] = s

    @pl.when(b > 0)
    def _():
        s_ref[...] = s_ref[...] + s


def _scale_shift(s_ref, hd, g_ref, be_ref, ntot, bn):
    s0 = 2.0 * s_ref[0:1] + bn * hd
    s1 = 2.0 * s_ref[1:2] + bn * hd * hd
    mean = s0 / ntot
    var = s1 / ntot - mean * mean
    scl = g_ref[...] * jax.lax.rsqrt(var + _EPS)
    sft = be_ref[...] - mean * scl
    return scl, sft


def _act(y):
    return jnp.maximum(y, 0.01 * y)


def _mom_update(m_ref, sd_ref, xc, r2_ref, t2_ref):
    b = pl.program_id(0)
    d = jnp.abs(jnp.dot(r2_ref[...], xc, preferred_element_type=_F32)
                - jnp.dot(t2_ref[...], xc, preferred_element_type=_F32))
    m = jax.lax.dot_general(d, d, (((0,), (0,)), ((), ())),
                            preferred_element_type=_F32)
    sd = jnp.dot(jnp.ones((1, d.shape[0]), _F32), d,
                 preferred_element_type=_F32)

    @pl.when(b == 0)
    def _():
        m_ref[...] = m
        sd_ref[...] = sd

    @pl.when(b > 0)
    def _():
        m_ref[...] = m_ref[...] + m
        sd_ref[...] = sd_ref[...] + sd


def _k_mom(x_ref, r2_ref, t2_ref, m_ref, sd_ref):
    _mom_update(m_ref, sd_ref, x_ref[0], r2_ref, t2_ref)


def _k_l01(x_ref, r2_ref, t2_ref, a_ref, c_ref, w0_ref, w1_ref,
           h_ref, s_ref, hd_ref, *, p2):
    x = x_ref[0]
    d = jnp.abs(jnp.dot(r2_ref[...], x, preferred_element_type=_F32)
                - jnp.dot(t2_ref[...], x, preferred_element_type=_F32))
    h0 = jnp.dot(d, w0_ref[...], preferred_element_type=_F32)
    t = _act(h0 * a_ref[...] + c_ref[...]) * _rowmask(p2, d.shape[0])
    h1 = jnp.dot(t, w1_ref[...], preferred_element_type=_F32)
    h_ref[0] = h1
    _stats_update(s_ref, h1)
    td = _act(c_ref[...])
    hd_ref[...] = jnp.dot(td, w1_ref[...], preferred_element_type=_F32)


def _k_mid2(h_ref, s_ref, hd_ref, g_ref, be_ref, w_ref,
            o_ref, so_ref, hdo_ref, *, ntot, bn, p2):
    scl, sft = _scale_shift(s_ref, hd_ref[...], g_ref, be_ref, ntot, bn)
    t = _act(h_ref[0] * scl + sft) * _rowmask(p2, h_ref.shape[1])
    h = jnp.dot(t, w_ref[...], preferred_element_type=_F32)
    o_ref[0] = h
    _stats_update(so_ref, h)
    td = _act(hd_ref[...] * scl + sft)
    hdo_ref[...] = jnp.dot(td, w_ref[...], preferred_element_type=_F32)


def _k_last2(h_ref, s_ref, hd_ref, g_ref, be_ref, w_ref, r2_ref, t2_ref,
             o_ref, *, n, ntot, bn, p2):
    scl, sft = _scale_shift(s_ref, hd_ref[...], g_ref, be_ref, ntot, bn)
    t = _act(h_ref[0] * scl + sft) * _rowmask(p2, h_ref.shape[1])
    l = jnp.sum(t * w_ref[...], axis=-1, keepdims=True)
    l1 = jax.lax.dot_general(r2_ref[...], l * t2_ref[...],
                             (((0,), (0,)), ((), ())),
                             preferred_element_type=_F32)
    l2 = jax.lax.dot_general(t2_ref[...], l * r2_ref[...],
                             (((0,), (0,)), ((), ())),
                             preferred_element_type=_F32)
    td = _act(hd_ref[...] * scl + sft)
    ld = jnp.sum(td * w_ref[...])
    eye = (_iota((n, n), 0) == _iota((n, n), 1)).astype(_F32)
    lg = l1 + l2 + ld * eye
    e = jnp.exp(lg)
    o_ref[0] = e / jnp.sum(e, axis=-1, keepdims=True)


def _k_gcn(an_ref, ao_ref, x_ref, w_ref, b_ref, o_ref, *, logsm):
    x = x_ref[0]
    d = x.shape[-1]
    x1 = jnp.dot(an_ref[0], x, preferred_element_type=_F32)
    x2 = jnp.dot(ao_ref[0], x, preferred_element_type=_F32)
    w = w_ref[...]
    o = (jnp.dot(x1, w[:d], preferred_element_type=_F32)
         + jnp.dot(x2, w[d:], preferred_element_type=_F32) + b_ref[...])
    if logsm:
        mx = jnp.max(o, axis=-1, keepdims=True)
        o = o - mx
        o = o - jnp.log(jnp.sum(jnp.exp(o), axis=-1, keepdims=True))
    o_ref[0] = o


def _k_gcn_mom(an_ref, ao_ref, x_ref, w_ref, b_ref, r2_ref, t2_ref,
               o_ref, m_ref, sd_ref):
    x = x_ref[0]
    d = x.shape[-1]
    x1 = jnp.dot(an_ref[0], x, preferred_element_type=_F32)
    x2 = jnp.dot(ao_ref[0], x, preferred_element_type=_F32)
    w = w_ref[...]
    o = (jnp.dot(x1, w[:d], preferred_element_type=_F32)
         + jnp.dot(x2, w[d:], preferred_element_type=_F32) + b_ref[...])
    o_ref[0] = o
    _mom_update(m_ref, sd_ref, o, r2_ref, t2_ref)


def _k_gcn_mom_cat(an_ref, ao_ref, x_ref, w_ref, b_ref, org_ref, r2_ref,
                   t2_ref, o_ref, m_ref, sd_ref):
    x = x_ref[0]
    d = x.shape[-1]
    x1 = jnp.dot(an_ref[0], x, preferred_element_type=_F32)
    x2 = jnp.dot(ao_ref[0], x, preferred_element_type=_F32)
    w = w_ref[...]
    o = (jnp.dot(x1, w[:d], preferred_element_type=_F32)
         + jnp.dot(x2, w[d:], preferred_element_type=_F32) + b_ref[...])
    o_ref[0] = o
    xc = jnp.concatenate([o, org_ref[0]], axis=-1)
    _mom_update(m_ref, sd_ref, xc, r2_ref, t2_ref)


def _k_pool(x_ref, a_ref, w_ref, b_ref, r2_ref, t2_ref, na_ref, nx_ref, p_ref, m_ref, sd_ref, *, ns, kint, newn):
    x = x_ref[0]
    adj = a_ref[0]
    n = x.shape[0]
    raw = (jnp.sum(x * w_ref[...], axis=-1, keepdims=True) + b_ref[0, 0]) / 100.0
    sc = jax.nn.sigmoid(raw)
    sr = jnp.transpose(sc)
    rowi = _iota((n, n), 0)
    colj = _iota((n, n), 1)
    m2 = ((sc > sr) | ((sc == sr) & (rowi < colj))) & (rowi < ns)
    rank = jnp.sum(m2.astype(_F32), axis=0, keepdims=True)
    cj = _iota((1, n), 1)
    rank = jnp.where(rank < kint, rank, 1e9)
    rankf = jnp.where(cj < ns, rank, (kint + cj - ns).astype(_F32))
    pm = (_iota((newn, n), 0).astype(_F32) == rankf).astype(_F32)
    vals = jnp.dot(pm, sc, preferred_element_type=_F32)
    nx = jnp.dot(pm, x, preferred_element_type=_F32) * vals
    nx_ref[0] = nx
    _mom_update(m_ref, sd_ref, nx, r2_ref, t2_ref)
    ta = jnp.dot(pm, adj, preferred_element_type=_F32)
    na_ref[0] = jax.lax.dot_general(ta, pm, (((1,), (1,)), ((), ())),
                                    preferred_element_type=_F32)
    p_ref[0] = pm


def _k_unpool(p_ref, x_ref, d_ref, r2_ref, t2_ref, o_ref, m_ref, sd_ref):
    o = jax.lax.dot_general(p_ref[0], x_ref[0],
                            (((0,), (0,)), ((), ())),
                            preferred_element_type=_F32) + d_ref[0]
    o_ref[0] = o
    _mom_update(m_ref, sd_ref, o, r2_ref, t2_ref)


def _full(shape):
    nd = len(shape)
    return pl.BlockSpec(shape, lambda b: (0,) * nd)


def _perb(shape):
    nd = len(shape)
    return pl.BlockSpec((1,) + shape, lambda b: (b,) + (0,) * nd)


def _tri_jnp(n):
    p2, p2p, r2np, t2np = _tri_np(n)
    return p2, p2p, jnp.asarray(r2np), jnp.asarray(t2np)


def _pallas_mlp(p, x, ms=None):
    b, n, d = x.shape
    ntot = float(b * n * n)
    bn = float(b * n)
    p2, p2p, r2, t2 = _tri_jnp(n)
    if ms is None:
        mom, sd = pl.pallas_call(
            _k_mom,
            grid=(b,),
            in_specs=[_perb((n, d)), _full((p2p, n)), _full((p2p, n))],
            out_specs=[_full((d, d)), _full((1, d))],
            out_shape=[jax.ShapeDtypeStruct((d, d), _F32),
                       jax.ShapeDtypeStruct((1, d), _F32)],
        )(x, r2, t2)
    else:
        mom, sd = ms
    mean0 = (2.0 * sd / ntot) @ p['w0']
    ex2 = jnp.sum(p['w0'] * ((2.0 * mom / ntot) @ p['w0']), axis=0)[None]
    var0 = ex2 - mean0 * mean0
    scl0 = p['g0'][None] * jax.lax.rsqrt(var0 + _EPS)
    sft0 = p['be0'][None] - mean0 * scl0
    h, s, hd = pl.pallas_call(
        functools.partial(_k_l01, p2=p2),
        grid=(b,),
        in_specs=[_perb((n, d)), _full((p2p, n)), _full((p2p, n)),
                  _full((1, 192)), _full((1, 192)),
                  _full((d, 192)), _full((192, 192))],
        out_specs=[_perb((p2p, 192)), _full((2, 192)), _full((1, 192))],
        out_shape=[jax.ShapeDtypeStruct((b, p2p, 192), _F32),
                   jax.ShapeDtypeStruct((2, 192), _F32),
                   jax.ShapeDtypeStruct((1, 192), _F32)],
    )(x, r2, t2, scl0, sft0, p['w0'], p['w1'])
    chs = [d, 192, 192, 96, 96]
    for i in range(2, 4):
        cin, cout = chs[i], chs[i + 1]
        h, s, hd = pl.pallas_call(
            functools.partial(_k_mid2, ntot=ntot, bn=bn, p2=p2),
            grid=(b,),
            in_specs=[_perb((p2p, cin)), _full((2, cin)), _full((1, cin)),
                      _full((1, cin)), _full((1, cin)), _full((cin, cout))],
            out_specs=[_perb((p2p, cout)), _full((2, cout)), _full((1, cout))],
            out_shape=[jax.ShapeDtypeStruct((b, p2p, cout), _F32),
                       jax.ShapeDtypeStruct((2, cout), _F32),
                       jax.ShapeDtypeStruct((1, cout), _F32)],
        )(h, s, hd, p['g%d' % (i - 1)][None], p['be%d' % (i - 1)][None],
          p['w%d' % i])
    return pl.pallas_call(
        functools.partial(_k_last2, n=n, ntot=ntot, bn=bn, p2=p2),
        grid=(b,),
        in_specs=[_perb((p2p, 96)), _full((2, 96)), _full((1, 96)),
                  _full((1, 96)), _full((1, 96)), _full((1, 96)),
                  _full((p2p, n)), _full((p2p, n))],
        out_specs=_perb((n, n)),
        out_shape=jax.ShapeDtypeStruct((b, n, n), _F32),
    )(h, s, hd, p['g3'][None], p['be3'][None], jnp.transpose(p['w4']),
      r2, t2)


def _pallas_gcn(p, a_new, a_old, x, logsm=False):
    b, n, d = x.shape
    dout = p['w'].shape[1]
    return pl.pallas_call(
        functools.partial(_k_gcn, logsm=logsm),
        grid=(b,),
        in_specs=[_perb((n, n)), _perb((n, n)), _perb((n, d)),
                  _full((2 * d, dout)), _full((1, dout))],
        out_specs=_perb((n, dout)),
        out_shape=jax.ShapeDtypeStruct((b, n, dout), _F32),
    )(a_new, a_old, x, p['w'], p['b'][None])


def _pallas_gcn_mom(p, a_new, a_old, x, org=None):
    b, n, d = x.shape
    dout = p['w'].shape[1]
    dc = dout + (org.shape[-1] if org is not None else 0)
    p2, p2p, r2, t2 = _tri_jnp(n)
    base = [_perb((n, n)), _perb((n, n)), _perb((n, d)),
            _full((2 * d, dout)), _full((1, dout))]
    outs = [_perb((n, dout)), _full((dc, dc)), _full((1, dc))]
    oshp = [jax.ShapeDtypeStruct((b, n, dout), _F32),
            jax.ShapeDtypeStruct((dc, dc), _F32),
            jax.ShapeDtypeStruct((1, dc), _F32)]
    if org is None:
        o, m, sd = pl.pallas_call(
            _k_gcn_mom,
            grid=(b,),
            in_specs=base + [_full((p2p, n)), _full((p2p, n))],
            out_specs=outs,
            out_shape=oshp,
        )(a_new, a_old, x, p['w'], p['b'][None], r2, t2)
    else:
        o, m, sd = pl.pallas_call(
            _k_gcn_mom_cat,
            grid=(b,),
            in_specs=base + [_perb((n, org.shape[-1])),
                             _full((p2p, n)), _full((p2p, n))],
            out_specs=outs,
            out_shape=oshp,
        )(a_new, a_old, x, p['w'], p['b'][None], org, r2, t2)
    return o, (m, sd)


def _pallas_pool(p, k, adj, x):
    b, n, d = x.shape
    ns = n - _NQ
    kint = int(k * ns)
    newn = kint + _NQ
    p2, p2p, r2, t2 = _tri_jnp(newn)
    na, nx, pm, m, sd = pl.pallas_call(
        functools.partial(_k_pool, ns=ns, kint=kint, newn=newn),
        grid=(b,),
        in_specs=[_perb((n, d)), _perb((n, n)), _full((1, d)), _full((1, 1)),
                  _full((p2p, newn)), _full((p2p, newn))],
        out_specs=[_perb((newn, newn)), _perb((newn, d)), _perb((newn, n)),
                   _full((d, d)), _full((1, d))],
        out_shape=[jax.ShapeDtypeStruct((b, newn, newn), _F32),
                   jax.ShapeDtypeStruct((b, newn, d), _F32),
                   jax.ShapeDtypeStruct((b, newn, n), _F32),
                   jax.ShapeDtypeStruct((d, d), _F32),
                   jax.ShapeDtypeStruct((1, d), _F32)],
    )(x, adj, jnp.transpose(p['w']), p['b'].reshape(1, 1), r2, t2)
    return na, nx, pm, (m, sd)


def _pallas_unpool_add(pm, xp, down):
    b, newn, n = pm.shape
    d = xp.shape[-1]
    p2, p2p, r2, t2 = _tri_jnp(n)
    o, m, sd = pl.pallas_call(
        _k_unpool,
        grid=(b,),
        in_specs=[_perb((newn, n)), _perb((newn, d)), _perb((n, d)),
                  _full((p2p, n)), _full((p2p, n))],
        out_specs=[_perb((n, d)), _full((d, d)), _full((1, d))],
        out_shape=[jax.ShapeDtypeStruct((b, n, d), _F32),
                   jax.ShapeDtypeStruct((d, d), _F32),
                   jax.ShapeDtypeStruct((1, d), _F32)],
    )(pm, xp, down, r2, t2)
    return o, (m, sd)


def kernel(A_init, X, params):
    org_x = X
    a_new = _pallas_mlp(params['start_mlp'], X)
    x, ms = _pallas_gcn_mom(params['start_gcn'], a_new, A_init, X)
    adj, downs, pms = [], [], []
    for i in range(_LN):
        a_old = a_new
        a_new = _pallas_mlp(params['down_mlp_%d' % i], x, ms)
        x = _pallas_gcn(params['down_gcn_%d' % i], a_new, a_old, x)
        adj.append(a_new)
        downs.append(x)
        a_new, x, pm, ms = _pallas_pool(params['pool_%d' % i], _KS[i],
                                        a_new, x)
        pms.append(pm)
    a_old = a_new
    a_new = _pallas_mlp(params['bottom_mlp'], x, ms)
    x = _pallas_gcn(params['bottom_gcn'], a_new, a_old, x)
    for i in range(_LN):
        u = _LN - 1 - i
        a_old = adj[u]
        x, ms = _pallas_unpool_add(pms[u], x, downs[u])
        a_new = _pallas_mlp(params['up_mlp_%d' % u], x, ms)
        if u > 0:
            x = _pallas_gcn(params['up_gcn_%d' % u], a_new, a_old, x)
        else:
            x, ms = _pallas_gcn_mom(params['up_gcn_0'], a_new, a_old, x,
                                    org=org_x)
    x = jnp.concatenate([x, org_x], axis=-1)
    a_old = a_new
    a_new = _pallas_mlp(params['out_mlp'], x, ms)
    return _pallas_gcn(params['out_gcn'], a_new, a_old, x, logsm=True)


# speedup vs baseline: 1.8567x; 1.1574x over previous
import functools

import jax
import jax.numpy as jnp
import numpy as np
from jax.experimental import pallas as pl

_EPS = 1e-5
_NQ = 16
_KS = [0.5, 0.5]
_LN = 2
_F32 = jnp.float32


def _iota(shape, dim):
    return jax.lax.broadcasted_iota(jnp.int32, shape, dim)


@functools.lru_cache(maxsize=None)
def _tri_np(n):
    p2 = n * (n - 1) // 2
    p2p = -(-p2 // 8) * 8
    iu, ju = np.triu_indices(n, 1)
    r2 = np.zeros((p2p, n), np.float32)
    t2 = np.zeros((p2p, n), np.float32)
    r2[np.arange(p2), iu] = 1.0
    t2[np.arange(p2), ju] = 1.0
    return p2, p2p, r2, t2


def _rowmask(p2, p2p, nb):
    ii = _iota((nb * p2p, 1), 0)
    return ((ii % p2p) < p2).astype(_F32)


def _stats_update(s_ref, h):
    b = pl.program_id(0)
    ones = jnp.ones((1, h.shape[0]), _F32)
    s0 = jnp.dot(ones, h, preferred_element_type=_F32)
    s1 = jnp.dot(ones, h * h, preferred_element_type=_F32)
    s = jnp.concatenate([s0, s1], axis=0)

    @pl.when(b == 0)
    def _():
        s_ref[...] = s

    @pl.when(b > 0)
    def _():
        s_ref[...] = s_ref[...] + s


def _scale_shift(s_ref, hd, g_ref, be_ref, ntot, bn):
    s0 = 2.0 * s_ref[0:1] + bn * hd
    s1 = 2.0 * s_ref[1:2] + bn * hd * hd
    mean = s0 / ntot
    var = s1 / ntot - mean * mean
    scl = g_ref[...] * jax.lax.rsqrt(var + _EPS)
    sft = be_ref[...] - mean * scl
    return scl, sft


def _act(y):
    return jnp.maximum(y, 0.01 * y)


def _mom_update(m_ref, sd_ref, xc, r2_ref, t2_ref):
    b = pl.program_id(0)
    d = jnp.abs(jnp.dot(r2_ref[...], xc, preferred_element_type=_F32)
                - jnp.dot(t2_ref[...], xc, preferred_element_type=_F32))
    m = jax.lax.dot_general(d, d, (((0,), (0,)), ((), ())),
                            preferred_element_type=_F32)
    sd = jnp.dot(jnp.ones((1, d.shape[0]), _F32), d,
                 preferred_element_type=_F32)

    @pl.when(b == 0)
    def _():
        m_ref[...] = m
        sd_ref[...] = sd

    @pl.when(b > 0)
    def _():
        m_ref[...] = m_ref[...] + m
        sd_ref[...] = sd_ref[...] + sd


def _k_mom(x_ref, r2_ref, t2_ref, m_ref, sd_ref):
    _mom_update(m_ref, sd_ref, x_ref[0], r2_ref, t2_ref)


def _k_l01(x_ref, r2_ref, t2_ref, a_ref, c_ref, w0_ref, w1_ref,
           h_ref, s_ref, hd_ref, *, p2, nb):
    r2 = r2_ref[...]
    t2 = t2_ref[...]
    d = jnp.concatenate(
        [jnp.abs(jnp.dot(r2, x_ref[0, bi], preferred_element_type=_F32)
                 - jnp.dot(t2, x_ref[0, bi], preferred_element_type=_F32))
         for bi in range(nb)], axis=0)
    h0 = jnp.dot(d, w0_ref[...], preferred_element_type=_F32)
    p2p = r2.shape[0]
    t = _act(h0 * a_ref[...] + c_ref[...]) * _rowmask(p2, p2p, nb)
    h1 = jnp.dot(t, w1_ref[...], preferred_element_type=_F32)
    h_ref[...] = h1.reshape(h_ref.shape)
    _stats_update(s_ref, h1)
    td = _act(c_ref[...])
    hd_ref[...] = jnp.dot(td, w1_ref[...], preferred_element_type=_F32)


def _k_mid2(h_ref, s_ref, hd_ref, g_ref, be_ref, w_ref,
            o_ref, so_ref, hdo_ref, *, ntot, bn, p2):
    nb, p2p = h_ref.shape[1], h_ref.shape[2]
    hin = h_ref[...].reshape(nb * p2p, h_ref.shape[3])
    scl, sft = _scale_shift(s_ref, hd_ref[...], g_ref, be_ref, ntot, bn)
    t = _act(hin * scl + sft) * _rowmask(p2, p2p, nb)
    h = jnp.dot(t, w_ref[...], preferred_element_type=_F32)
    o_ref[...] = h.reshape(o_ref.shape)
    _stats_update(so_ref, h)
    td = _act(hd_ref[...] * scl + sft)
    hdo_ref[...] = jnp.dot(td, w_ref[...], preferred_element_type=_F32)


def _k_last2(h_ref, s_ref, hd_ref, g_ref, be_ref, w_ref, r2_ref, t2_ref,
             o_ref, *, n, ntot, bn, p2):
    nb, p2p = h_ref.shape[1], h_ref.shape[2]
    hin = h_ref[...].reshape(nb * p2p, h_ref.shape[3])
    scl, sft = _scale_shift(s_ref, hd_ref[...], g_ref, be_ref, ntot, bn)
    t = _act(hin * scl + sft) * _rowmask(p2, p2p, nb)
    l = jnp.sum(t * w_ref[...], axis=-1, keepdims=True)
    td = _act(hd_ref[...] * scl + sft)
    ld = jnp.sum(td * w_ref[...])
    eye = (_iota((n, n), 0) == _iota((n, n), 1)).astype(_F32)
    r2 = r2_ref[...]
    t2 = t2_ref[...]
    for bi in range(nb):
        li = l[bi * p2p:(bi + 1) * p2p]
        l1 = jax.lax.dot_general(r2, li * t2, (((0,), (0,)), ((), ())),
                                 preferred_element_type=_F32)
        l2 = jax.lax.dot_general(t2, li * r2, (((0,), (0,)), ((), ())),
                                 preferred_element_type=_F32)
        e = jnp.exp(l1 + l2 + ld * eye)
        o_ref[0, bi] = e / jnp.sum(e, axis=-1, keepdims=True)


def _k_gcn(an_ref, ao_ref, x_ref, w_ref, b_ref, o_ref, *, logsm):
    x = x_ref[0]
    d = x.shape[-1]
    x1 = jnp.dot(an_ref[0], x, preferred_element_type=_F32)
    x2 = jnp.dot(ao_ref[0], x, preferred_element_type=_F32)
    w = w_ref[...]
    o = (jnp.dot(x1, w[:d], preferred_element_type=_F32)
         + jnp.dot(x2, w[d:], preferred_element_type=_F32) + b_ref[...])
    if logsm:
        mx = jnp.max(o, axis=-1, keepdims=True)
        o = o - mx
        o = o - jnp.log(jnp.sum(jnp.exp(o), axis=-1, keepdims=True))
    o_ref[0] = o


def _k_gcn_mom(an_ref, ao_ref, x_ref, w_ref, b_ref, r2_ref, t2_ref,
               o_ref, m_ref, sd_ref):
    x = x_ref[0]
    d = x.shape[-1]
    x1 = jnp.dot(an_ref[0], x, preferred_element_type=_F32)
    x2 = jnp.dot(ao_ref[0], x, preferred_element_type=_F32)
    w = w_ref[...]
    o = (jnp.dot(x1, w[:d], preferred_element_type=_F32)
         + jnp.dot(x2, w[d:], preferred_element_type=_F32) + b_ref[...])
    o_ref[0] = o
    _mom_update(m_ref, sd_ref, o, r2_ref, t2_ref)


def _k_gcn_mom_cat(an_ref, ao_ref, x_ref, w_ref, b_ref, org_ref, r2_ref,
                   t2_ref, o_ref, m_ref, sd_ref):
    x = x_ref[0]
    d = x.shape[-1]
    x1 = jnp.dot(an_ref[0], x, preferred_element_type=_F32)
    x2 = jnp.dot(ao_ref[0], x, preferred_element_type=_F32)
    w = w_ref[...]
    o = (jnp.dot(x1, w[:d], preferred_element_type=_F32)
         + jnp.dot(x2, w[d:], preferred_element_type=_F32) + b_ref[...])
    o_ref[0] = o
    xc = jnp.concatenate([o, org_ref[0]], axis=-1)
    _mom_update(m_ref, sd_ref, xc, r2_ref, t2_ref)


def _k_pool(x_ref, a_ref, w_ref, b_ref, r2_ref, t2_ref, na_ref, nx_ref, p_ref, m_ref, sd_ref, *, ns, kint, newn):
    x = x_ref[0]
    adj = a_ref[0]
    n = x.shape[0]
    raw = (jnp.sum(x * w_ref[...], axis=-1, keepdims=True) + b_ref[0, 0]) / 100.0
    sc = jax.nn.sigmoid(raw)
    sr = jnp.transpose(sc)
    rowi = _iota((n, n), 0)
    colj = _iota((n, n), 1)
    m2 = ((sc > sr) | ((sc == sr) & (rowi < colj))) & (rowi < ns)
    rank = jnp.sum(m2.astype(_F32), axis=0, keepdims=True)
    cj = _iota((1, n), 1)
    rank = jnp.where(rank < kint, rank, 1e9)
    rankf = jnp.where(cj < ns, rank, (kint + cj - ns).astype(_F32))
    pm = (_iota((newn, n), 0).astype(_F32) == rankf).astype(_F32)
    vals = jnp.dot(pm, sc, preferred_element_type=_F32)
    nx = jnp.dot(pm, x, preferred_element_type=_F32) * vals
    nx_ref[0] = nx
    _mom_update(m_ref, sd_ref, nx, r2_ref, t2_ref)
    ta = jnp.dot(pm, adj, preferred_element_type=_F32)
    na_ref[0] = jax.lax.dot_general(ta, pm, (((1,), (1,)), ((), ())),
                                    preferred_element_type=_F32)
    p_ref[0] = pm


def _k_unpool(p_ref, x_ref, d_ref, r2_ref, t2_ref, o_ref, m_ref, sd_ref):
    o = jax.lax.dot_general(p_ref[0], x_ref[0],
                            (((0,), (0,)), ((), ())),
                            preferred_element_type=_F32) + d_ref[0]
    o_ref[0] = o
    _mom_update(m_ref, sd_ref, o, r2_ref, t2_ref)


def _full(shape):
    nd = len(shape)
    return pl.BlockSpec(shape, lambda b: (0,) * nd)


def _perb(shape):
    nd = len(shape)
    return pl.BlockSpec((1,) + shape, lambda b: (b,) + (0,) * nd)


def _tri_jnp(n):
    p2, p2p, r2np, t2np = _tri_np(n)
    return p2, p2p, jnp.asarray(r2np), jnp.asarray(t2np)


def _pallas_mlp(p, x, ms=None):
    b, n, d = x.shape
    ntot = float(b * n * n)
    bn = float(b * n)
    p2, p2p, r2, t2 = _tri_jnp(n)
    if ms is None:
        mom, sd = pl.pallas_call(
            _k_mom,
            grid=(b,),
            in_specs=[_perb((n, d)), _full((p2p, n)), _full((p2p, n))],
            out_specs=[_full((d, d)), _full((1, d))],
            out_shape=[jax.ShapeDtypeStruct((d, d), _F32),
                       jax.ShapeDtypeStruct((1, d), _F32)],
        )(x, r2, t2)
    else:
        mom, sd = ms
    mean0 = (2.0 * sd / ntot) @ p['w0']
    ex2 = jnp.sum(p['w0'] * ((2.0 * mom / ntot) @ p['w0']), axis=0)[None]
    var0 = ex2 - mean0 * mean0
    scl0 = p['g0'][None] * jax.lax.rsqrt(var0 + _EPS)
    sft0 = p['be0'][None] - mean0 * scl0
    nb = 4
    gb = b // nb
    h, s, hd = pl.pallas_call(
        functools.partial(_k_l01, p2=p2, nb=nb),
        grid=(gb,),
        in_specs=[_perb((nb, n, d)), _full((p2p, n)), _full((p2p, n)),
                  _full((1, 192)), _full((1, 192)),
                  _full((d, 192)), _full((192, 192))],
        out_specs=[_perb((nb, p2p, 192)), _full((2, 192)), _full((1, 192))],
        out_shape=[jax.ShapeDtypeStruct((gb, nb, p2p, 192), _F32),
                   jax.ShapeDtypeStruct((2, 192), _F32),
                   jax.ShapeDtypeStruct((1, 192), _F32)],
    )(x.reshape(gb, nb, n, d), r2, t2, scl0, sft0, p['w0'], p['w1'])
    chs = [d, 192, 192, 96, 96]
    for i in range(2, 4):
        cin, cout = chs[i], chs[i + 1]
        h, s, hd = pl.pallas_call(
            functools.partial(_k_mid2, ntot=ntot, bn=bn, p2=p2),
            grid=(gb,),
            in_specs=[_perb((nb, p2p, cin)), _full((2, cin)), _full((1, cin)),
                      _full((1, cin)), _full((1, cin)), _full((cin, cout))],
            out_specs=[_perb((nb, p2p, cout)), _full((2, cout)),
                       _full((1, cout))],
            out_shape=[jax.ShapeDtypeStruct((gb, nb, p2p, cout), _F32),
                       jax.ShapeDtypeStruct((2, cout), _F32),
                       jax.ShapeDtypeStruct((1, cout), _F32)],
        )(h, s, hd, p['g%d' % (i - 1)][None], p['be%d' % (i - 1)][None],
          p['w%d' % i])
    a4 = pl.pallas_call(
        functools.partial(_k_last2, n=n, ntot=ntot, bn=bn, p2=p2),
        grid=(gb,),
        in_specs=[_perb((nb, p2p, 96)), _full((2, 96)), _full((1, 96)),
                  _full((1, 96)), _full((1, 96)), _full((1, 96)),
                  _full((p2p, n)), _full((p2p, n))],
        out_specs=_perb((nb, n, n)),
        out_shape=jax.ShapeDtypeStruct((gb, nb, n, n), _F32),
    )(h, s, hd, p['g3'][None], p['be3'][None], jnp.transpose(p['w4']),
      r2, t2)
    return a4.reshape(b, n, n)


def _pallas_gcn(p, a_new, a_old, x, logsm=False):
    b, n, d = x.shape
    dout = p['w'].shape[1]
    return pl.pallas_call(
        functools.partial(_k_gcn, logsm=logsm),
        grid=(b,),
        in_specs=[_perb((n, n)), _perb((n, n)), _perb((n, d)),
                  _full((2 * d, dout)), _full((1, dout))],
        out_specs=_perb((n, dout)),
        out_shape=jax.ShapeDtypeStruct((b, n, dout), _F32),
    )(a_new, a_old, x, p['w'], p['b'][None])


def _pallas_gcn_mom(p, a_new, a_old, x, org=None):
    b, n, d = x.shape
    dout = p['w'].shape[1]
    dc = dout + (org.shape[-1] if org is not None else 0)
    p2, p2p, r2, t2 = _tri_jnp(n)
    base = [_perb((n, n)), _perb((n, n)), _perb((n, d)),
            _full((2 * d, dout)), _full((1, dout))]
    outs = [_perb((n, dout)), _full((dc, dc)), _full((1, dc))]
    oshp = [jax.ShapeDtypeStruct((b, n, dout), _F32),
            jax.ShapeDtypeStruct((dc, dc), _F32),
            jax.ShapeDtypeStruct((1, dc), _F32)]
    if org is None:
        o, m, sd = pl.pallas_call(
            _k_gcn_mom,
            grid=(b,),
            in_specs=base + [_full((p2p, n)), _full((p2p, n))],
            out_specs=outs,
            out_shape=oshp,
        )(a_new, a_old, x, p['w'], p['b'][None], r2, t2)
    else:
        o, m, sd = pl.pallas_call(
            _k_gcn_mom_cat,
            grid=(b,),
            in_specs=base + [_perb((n, org.shape[-1])),
                             _full((p2p, n)), _full((p2p, n))],
            out_specs=outs,
            out_shape=oshp,
        )(a_new, a_old, x, p['w'], p['b'][None], org, r2, t2)
    return o, (m, sd)


def _pallas_pool(p, k, adj, x):
    b, n, d = x.shape
    ns = n - _NQ
    kint = int(k * ns)
    newn = kint + _NQ
    p2, p2p, r2, t2 = _tri_jnp(newn)
    na, nx, pm, m, sd = pl.pallas_call(
        functools.partial(_k_pool, ns=ns, kint=kint, newn=newn),
        grid=(b,),
        in_specs=[_perb((n, d)), _perb((n, n)), _full((1, d)), _full((1, 1)),
                  _full((p2p, newn)), _full((p2p, newn))],
        out_specs=[_perb((newn, newn)), _perb((newn, d)), _perb((newn, n)),
                   _full((d, d)), _full((1, d))],
        out_shape=[jax.ShapeDtypeStruct((b, newn, newn), _F32),
                   jax.ShapeDtypeStruct((b, newn, d), _F32),
                   jax.ShapeDtypeStruct((b, newn, n), _F32),
                   jax.ShapeDtypeStruct((d, d), _F32),
                   jax.ShapeDtypeStruct((1, d), _F32)],
    )(x, adj, jnp.transpose(p['w']), p['b'].reshape(1, 1), r2, t2)
    return na, nx, pm, (m, sd)


def _pallas_unpool_add(pm, xp, down):
    b, newn, n = pm.shape
    d = xp.shape[-1]
    p2, p2p, r2, t2 = _tri_jnp(n)
    o, m, sd = pl.pallas_call(
        _k_unpool,
        grid=(b,),
        in_specs=[_perb((newn, n)), _perb((newn, d)), _perb((n, d)),
                  _full((p2p, n)), _full((p2p, n))],
        out_specs=[_perb((n, d)), _full((d, d)), _full((1, d))],
        out_shape=[jax.ShapeDtypeStruct((b, n, d), _F32),
                   jax.ShapeDtypeStruct((d, d), _F32),
                   jax.ShapeDtypeStruct((1, d), _F32)],
    )(pm, xp, down, r2, t2)
    return o, (m, sd)


def kernel(A_init, X, params):
    org_x = X
    a_new = _pallas_mlp(params['start_mlp'], X)
    x, ms = _pallas_gcn_mom(params['start_gcn'], a_new, A_init, X)
    adj, downs, pms = [], [], []
    for i in range(_LN):
        a_old = a_new
        a_new = _pallas_mlp(params['down_mlp_%d' % i], x, ms)
        x = _pallas_gcn(params['down_gcn_%d' % i], a_new, a_old, x)
        adj.append(a_new)
        downs.append(x)
        a_new, x, pm, ms = _pallas_pool(params['pool_%d' % i], _KS[i],
                                        a_new, x)
        pms.append(pm)
    a_old = a_new
    a_new = _pallas_mlp(params['bottom_mlp'], x, ms)
    x = _pallas_gcn(params['bottom_gcn'], a_new, a_old, x)
    for i in range(_LN):
        u = _LN - 1 - i
        a_old = adj[u]
        x, ms = _pallas_unpool_add(pms[u], x, downs[u])
        a_new = _pallas_mlp(params['up_mlp_%d' % u], x, ms)
        if u > 0:
            x = _pallas_gcn(params['up_gcn_%d' % u], a_new, a_old, x)
        else:
            x, ms = _pallas_gcn_mom(params['up_gcn_0'], a_new, a_old, x,
                                    org=org_x)
    x = jnp.concatenate([x, org_x], axis=-1)
    a_old = a_new
    a_new = _pallas_mlp(params['out_mlp'], x, ms)
    return _pallas_gcn(params['out_gcn'], a_new, a_old, x, logsm=True)


# speedup vs baseline: 2.1769x; 1.1725x over previous
import functools

import jax
import jax.numpy as jnp
import numpy as np
from jax.experimental import pallas as pl

_EPS = 1e-5
_NQ = 16
_KS = [0.5, 0.5]
_LN = 2
_F32 = jnp.float32


def _iota(shape, dim):
    return jax.lax.broadcasted_iota(jnp.int32, shape, dim)


@functools.lru_cache(maxsize=None)
def _tri_np(n):
    p2 = n * (n - 1) // 2
    p2p = -(-p2 // 8) * 8
    iu, ju = np.triu_indices(n, 1)
    r2 = np.zeros((p2p, n), np.float32)
    t2 = np.zeros((p2p, n), np.float32)
    r2[np.arange(p2), iu] = 1.0
    t2[np.arange(p2), ju] = 1.0
    return p2, p2p, r2, t2


def _rowmask(p2, p2p, nb):
    ii = _iota((nb * p2p, 1), 0)
    return ((ii % p2p) < p2).astype(_F32)


def _stats_update(s_ref, h):
    b = pl.program_id(0)
    ones = jnp.ones((1, h.shape[0]), _F32)
    s0 = jnp.dot(ones, h, preferred_element_type=_F32)
    s1 = jnp.dot(ones, h * h, preferred_element_type=_F32)
    s = jnp.concatenate([s0, s1], axis=0)

    @pl.when(b == 0)
    def _():
        s_ref[...] = s

    @pl.when(b > 0)
    def _():
        s_ref[...] = s_ref[...] + s


def _scale_shift(s_ref, hd, g_ref, be_ref, ntot, bn):
    s0 = 2.0 * s_ref[0:1] + bn * hd
    s1 = 2.0 * s_ref[1:2] + bn * hd * hd
    mean = s0 / ntot
    var = s1 / ntot - mean * mean
    scl = g_ref[...] * jax.lax.rsqrt(var + _EPS)
    sft = be_ref[...] - mean * scl
    return scl, sft


def _act(y):
    return jnp.maximum(y, 0.01 * y)


def _mom_update(m_ref, sd_ref, xcs, r2_ref, t2_ref):
    b = pl.program_id(0)
    d = jnp.concatenate(
        [jnp.abs(jnp.dot(r2_ref[...], xc, preferred_element_type=_F32)
                 - jnp.dot(t2_ref[...], xc, preferred_element_type=_F32))
         for xc in xcs], axis=0)
    m = jax.lax.dot_general(d, d, (((0,), (0,)), ((), ())),
                            preferred_element_type=_F32)
    sd = jnp.dot(jnp.ones((1, d.shape[0]), _F32), d,
                 preferred_element_type=_F32)

    @pl.when(b == 0)
    def _():
        m_ref[...] = m
        sd_ref[...] = sd

    @pl.when(b > 0)
    def _():
        m_ref[...] = m_ref[...] + m
        sd_ref[...] = sd_ref[...] + sd


def _k_mom(x_ref, r2_ref, t2_ref, m_ref, sd_ref, *, nb):
    _mom_update(m_ref, sd_ref, [x_ref[0, bi] for bi in range(nb)],
                r2_ref, t2_ref)


def _k_l01(x_ref, r2_ref, t2_ref, a_ref, c_ref, w0_ref, w1_ref,
           h_ref, s_ref, hd_ref, *, p2, nb):
    r2 = r2_ref[...]
    t2 = t2_ref[...]
    d = jnp.concatenate(
        [jnp.abs(jnp.dot(r2, x_ref[0, bi], preferred_element_type=_F32)
                 - jnp.dot(t2, x_ref[0, bi], preferred_element_type=_F32))
         for bi in range(nb)], axis=0)
    h0 = jnp.dot(d, w0_ref[...], preferred_element_type=_F32)
    p2p = r2.shape[0]
    t = _act(h0 * a_ref[...] + c_ref[...]) * _rowmask(p2, p2p, nb)
    h1 = jnp.dot(t, w1_ref[...], preferred_element_type=_F32)
    h_ref[...] = h1.reshape(h_ref.shape)
    _stats_update(s_ref, h1)
    td = _act(c_ref[...])
    hd_ref[...] = jnp.dot(td, w1_ref[...], preferred_element_type=_F32)


def _k_mid2(h_ref, s_ref, hd_ref, g_ref, be_ref, w_ref,
            o_ref, so_ref, hdo_ref, *, ntot, bn, p2):
    nb, p2p = h_ref.shape[1], h_ref.shape[2]
    hin = h_ref[...].reshape(nb * p2p, h_ref.shape[3])
    scl, sft = _scale_shift(s_ref, hd_ref[...], g_ref, be_ref, ntot, bn)
    t = _act(hin * scl + sft) * _rowmask(p2, p2p, nb)
    h = jnp.dot(t, w_ref[...], preferred_element_type=_F32)
    o_ref[...] = h.reshape(o_ref.shape)
    _stats_update(so_ref, h)
    td = _act(hd_ref[...] * scl + sft)
    hdo_ref[...] = jnp.dot(td, w_ref[...], preferred_element_type=_F32)


def _k_last2(h_ref, s_ref, hd_ref, g_ref, be_ref, w_ref, r2_ref, t2_ref,
             o_ref, *, n, ntot, bn, p2):
    nb, p2p = h_ref.shape[1], h_ref.shape[2]
    hin = h_ref[...].reshape(nb * p2p, h_ref.shape[3])
    scl, sft = _scale_shift(s_ref, hd_ref[...], g_ref, be_ref, ntot, bn)
    t = _act(hin * scl + sft) * _rowmask(p2, p2p, nb)
    l = jnp.sum(t * w_ref[...], axis=-1, keepdims=True)
    td = _act(hd_ref[...] * scl + sft)
    ld = jnp.sum(td * w_ref[...])
    eye = (_iota((n, n), 0) == _iota((n, n), 1)).astype(_F32)
    r2 = r2_ref[...]
    t2 = t2_ref[...]
    for bi in range(nb):
        li = l[bi * p2p:(bi + 1) * p2p]
        l1 = jax.lax.dot_general(r2, li * t2, (((0,), (0,)), ((), ())),
                                 preferred_element_type=_F32)
        l2 = jax.lax.dot_general(t2, li * r2, (((0,), (0,)), ((), ())),
                                 preferred_element_type=_F32)
        e = jnp.exp(l1 + l2 + ld * eye)
        o_ref[0, bi] = e / jnp.sum(e, axis=-1, keepdims=True)


def _gcn_one(an, ao, x, w, bia):
    d = x.shape[-1]
    x1 = jnp.dot(an, x, preferred_element_type=_F32)
    x2 = jnp.dot(ao, x, preferred_element_type=_F32)
    return (jnp.dot(x1, w[:d], preferred_element_type=_F32)
            + jnp.dot(x2, w[d:], preferred_element_type=_F32) + bia)


def _k_gcn(an_ref, ao_ref, x_ref, w_ref, b_ref, o_ref, *, logsm, nb):
    w = w_ref[...]
    bia = b_ref[...]
    for bi in range(nb):
        o = _gcn_one(an_ref[0, bi], ao_ref[0, bi], x_ref[0, bi], w, bia)
        if logsm:
            mx = jnp.max(o, axis=-1, keepdims=True)
            o = o - mx
            o = o - jnp.log(jnp.sum(jnp.exp(o), axis=-1, keepdims=True))
        o_ref[0, bi] = o


def _k_gcn_mom(an_ref, ao_ref, x_ref, w_ref, b_ref, r2_ref, t2_ref,
               o_ref, m_ref, sd_ref, *, nb):
    w = w_ref[...]
    bia = b_ref[...]
    os = []
    for bi in range(nb):
        o = _gcn_one(an_ref[0, bi], ao_ref[0, bi], x_ref[0, bi], w, bia)
        o_ref[0, bi] = o
        os.append(o)
    _mom_update(m_ref, sd_ref, os, r2_ref, t2_ref)


def _k_gcn_mom_cat(an_ref, ao_ref, x_ref, w_ref, b_ref, org_ref, r2_ref,
                   t2_ref, o_ref, m_ref, sd_ref, *, nb):
    w = w_ref[...]
    bia = b_ref[...]
    xcs = []
    for bi in range(nb):
        o = _gcn_one(an_ref[0, bi], ao_ref[0, bi], x_ref[0, bi], w, bia)
        o_ref[0, bi] = o
        xcs.append(jnp.concatenate([o, org_ref[0, bi]], axis=-1))
    _mom_update(m_ref, sd_ref, xcs, r2_ref, t2_ref)


def _k_pool(x_ref, a_ref, w_ref, b_ref, r2_ref, t2_ref, na_ref, nx_ref,
            p_ref, m_ref, sd_ref, *, ns, kint, newn, nb):
    wr = w_ref[...]
    b0 = b_ref[0, 0]
    nxs = []
    for bi in range(nb):
        x = x_ref[0, bi]
        adj = a_ref[0, bi]
        n = x.shape[0]
        raw = (jnp.sum(x * wr, axis=-1, keepdims=True) + b0) / 100.0
        sc = jax.nn.sigmoid(raw)
        sr = jnp.transpose(sc)
        rowi = _iota((n, n), 0)
        colj = _iota((n, n), 1)
        m2 = ((sc > sr) | ((sc == sr) & (rowi < colj))) & (rowi < ns)
        rank = jnp.sum(m2.astype(_F32), axis=0, keepdims=True)
        cj = _iota((1, n), 1)
        rank = jnp.where(rank < kint, rank, 1e9)
        rankf = jnp.where(cj < ns, rank, (kint + cj - ns).astype(_F32))
        pm = (_iota((newn, n), 0).astype(_F32) == rankf).astype(_F32)
        vals = jnp.dot(pm, sc, preferred_element_type=_F32)
        nx = jnp.dot(pm, x, preferred_element_type=_F32) * vals
        nx_ref[0, bi] = nx
        nxs.append(nx)
        ta = jnp.dot(pm, adj, preferred_element_type=_F32)
        na_ref[0, bi] = jax.lax.dot_general(ta, pm, (((1,), (1,)), ((), ())),
                                            preferred_element_type=_F32)
        p_ref[0, bi] = pm
    _mom_update(m_ref, sd_ref, nxs, r2_ref, t2_ref)


def _k_unpool(p_ref, x_ref, d_ref, r2_ref, t2_ref, o_ref, m_ref, sd_ref,
              *, nb):
    os = []
    for bi in range(nb):
        o = jax.lax.dot_general(p_ref[0, bi], x_ref[0, bi],
                                (((0,), (0,)), ((), ())),
                                preferred_element_type=_F32) + d_ref[0, bi]
        o_ref[0, bi] = o
        os.append(o)
    _mom_update(m_ref, sd_ref, os, r2_ref, t2_ref)


def _full(shape):
    nd = len(shape)
    return pl.BlockSpec(shape, lambda b: (0,) * nd)


def _perb(shape):
    nd = len(shape)
    return pl.BlockSpec((1,) + shape, lambda b: (b,) + (0,) * nd)


def _tri_jnp(n):
    p2, p2p, r2np, t2np = _tri_np(n)
    return p2, p2p, jnp.asarray(r2np), jnp.asarray(t2np)


def _pallas_mlp(p, x, ms=None):
    b, n, d = x.shape
    ntot = float(b * n * n)
    bn = float(b * n)
    p2, p2p, r2, t2 = _tri_jnp(n)
    if ms is None:
        mom, sd = pl.pallas_call(
            functools.partial(_k_mom, nb=4),
            grid=(b // 4,),
            in_specs=[_perb((4, n, d)), _full((p2p, n)), _full((p2p, n))],
            out_specs=[_full((d, d)), _full((1, d))],
            out_shape=[jax.ShapeDtypeStruct((d, d), _F32),
                       jax.ShapeDtypeStruct((1, d), _F32)],
        )(x.reshape(b // 4, 4, n, d), r2, t2)
    else:
        mom, sd = ms
    mean0 = (2.0 * sd / ntot) @ p['w0']
    ex2 = jnp.sum(p['w0'] * ((2.0 * mom / ntot) @ p['w0']), axis=0)[None]
    var0 = ex2 - mean0 * mean0
    scl0 = p['g0'][None] * jax.lax.rsqrt(var0 + _EPS)
    sft0 = p['be0'][None] - mean0 * scl0
    nb = 4
    gb = b // nb
    h, s, hd = pl.pallas_call(
        functools.partial(_k_l01, p2=p2, nb=nb),
        grid=(gb,),
        in_specs=[_perb((nb, n, d)), _full((p2p, n)), _full((p2p, n)),
                  _full((1, 192)), _full((1, 192)),
                  _full((d, 192)), _full((192, 192))],
        out_specs=[_perb((nb, p2p, 192)), _full((2, 192)), _full((1, 192))],
        out_shape=[jax.ShapeDtypeStruct((gb, nb, p2p, 192), _F32),
                   jax.ShapeDtypeStruct((2, 192), _F32),
                   jax.ShapeDtypeStruct((1, 192), _F32)],
    )(x.reshape(gb, nb, n, d), r2, t2, scl0, sft0, p['w0'], p['w1'])
    chs = [d, 192, 192, 96, 96]
    for i in range(2, 4):
        cin, cout = chs[i], chs[i + 1]
        h, s, hd = pl.pallas_call(
            functools.partial(_k_mid2, ntot=ntot, bn=bn, p2=p2),
            grid=(gb,),
            in_specs=[_perb((nb, p2p, cin)), _full((2, cin)), _full((1, cin)),
                      _full((1, cin)), _full((1, cin)), _full((cin, cout))],
            out_specs=[_perb((nb, p2p, cout)), _full((2, cout)),
                       _full((1, cout))],
            out_shape=[jax.ShapeDtypeStruct((gb, nb, p2p, cout), _F32),
                       jax.ShapeDtypeStruct((2, cout), _F32),
                       jax.ShapeDtypeStruct((1, cout), _F32)],
        )(h, s, hd, p['g%d' % (i - 1)][None], p['be%d' % (i - 1)][None],
          p['w%d' % i])
    a4 = pl.pallas_call(
        functools.partial(_k_last2, n=n, ntot=ntot, bn=bn, p2=p2),
        grid=(gb,),
        in_specs=[_perb((nb, p2p, 96)), _full((2, 96)), _full((1, 96)),
                  _full((1, 96)), _full((1, 96)), _full((1, 96)),
                  _full((p2p, n)), _full((p2p, n))],
        out_specs=_perb((nb, n, n)),
        out_shape=jax.ShapeDtypeStruct((gb, nb, n, n), _F32),
    )(h, s, hd, p['g3'][None], p['be3'][None], jnp.transpose(p['w4']),
      r2, t2)
    return a4.reshape(b, n, n)


def _pallas_gcn(p, a_new, a_old, x, logsm=False):
    b, n, d = x.shape
    dout = p['w'].shape[1]
    nb = 4
    gb = b // nb
    o = pl.pallas_call(
        functools.partial(_k_gcn, logsm=logsm, nb=nb),
        grid=(gb,),
        in_specs=[_perb((nb, n, n)), _perb((nb, n, n)), _perb((nb, n, d)),
                  _full((2 * d, dout)), _full((1, dout))],
        out_specs=_perb((nb, n, dout)),
        out_shape=jax.ShapeDtypeStruct((gb, nb, n, dout), _F32),
    )(a_new.reshape(gb, nb, n, n), a_old.reshape(gb, nb, n, n),
      x.reshape(gb, nb, n, d), p['w'], p['b'][None])
    return o.reshape(b, n, dout)


def _pallas_gcn_mom(p, a_new, a_old, x, org=None):
    b, n, d = x.shape
    dout = p['w'].shape[1]
    dc = dout + (org.shape[-1] if org is not None else 0)
    p2, p2p, r2, t2 = _tri_jnp(n)
    nb = 4
    gb = b // nb
    base = [_perb((nb, n, n)), _perb((nb, n, n)), _perb((nb, n, d)),
            _full((2 * d, dout)), _full((1, dout))]
    outs = [_perb((nb, n, dout)), _full((dc, dc)), _full((1, dc))]
    oshp = [jax.ShapeDtypeStruct((gb, nb, n, dout), _F32),
            jax.ShapeDtypeStruct((dc, dc), _F32),
            jax.ShapeDtypeStruct((1, dc), _F32)]
    an4 = a_new.reshape(gb, nb, n, n)
    ao4 = a_old.reshape(gb, nb, n, n)
    x4 = x.reshape(gb, nb, n, d)
    if org is None:
        o, m, sd = pl.pallas_call(
            functools.partial(_k_gcn_mom, nb=nb),
            grid=(gb,),
            in_specs=base + [_full((p2p, n)), _full((p2p, n))],
            out_specs=outs,
            out_shape=oshp,
        )(an4, ao4, x4, p['w'], p['b'][None], r2, t2)
    else:
        o, m, sd = pl.pallas_call(
            functools.partial(_k_gcn_mom_cat, nb=nb),
            grid=(gb,),
            in_specs=base + [_perb((nb, n, org.shape[-1])),
                             _full((p2p, n)), _full((p2p, n))],
            out_specs=outs,
            out_shape=oshp,
        )(an4, ao4, x4, p['w'], p['b'][None],
          org.reshape(gb, nb, n, org.shape[-1]), r2, t2)
    return o.reshape(b, n, dout), (m, sd)


def _pallas_pool(p, k, adj, x):
    b, n, d = x.shape
    ns = n - _NQ
    kint = int(k * ns)
    newn = kint + _NQ
    p2, p2p, r2, t2 = _tri_jnp(newn)
    nb = 4
    gb = b // nb
    na, nx, pm, m, sd = pl.pallas_call(
        functools.partial(_k_pool, ns=ns, kint=kint, newn=newn, nb=nb),
        grid=(gb,),
        in_specs=[_perb((nb, n, d)), _perb((nb, n, n)), _full((1, d)),
                  _full((1, 1)), _full((p2p, newn)), _full((p2p, newn))],
        out_specs=[_perb((nb, newn, newn)), _perb((nb, newn, d)),
                   _perb((nb, newn, n)), _full((d, d)), _full((1, d))],
        out_shape=[jax.ShapeDtypeStruct((gb, nb, newn, newn), _F32),
                   jax.ShapeDtypeStruct((gb, nb, newn, d), _F32),
                   jax.ShapeDtypeStruct((gb, nb, newn, n), _F32),
                   jax.ShapeDtypeStruct((d, d), _F32),
                   jax.ShapeDtypeStruct((1, d), _F32)],
    )(x.reshape(gb, nb, n, d), adj.reshape(gb, nb, n, n),
      jnp.transpose(p['w']), p['b'].reshape(1, 1), r2, t2)
    return (na.reshape(b, newn, newn), nx.reshape(b, newn, d),
            pm.reshape(b, newn, n), (m, sd))


def _pallas_unpool_add(pm, xp, down):
    b, newn, n = pm.shape
    d = xp.shape[-1]
    p2, p2p, r2, t2 = _tri_jnp(n)
    nb = 4
    gb = b // nb
    o, m, sd = pl.pallas_call(
        functools.partial(_k_unpool, nb=nb),
        grid=(gb,),
        in_specs=[_perb((nb, newn, n)), _perb((nb, newn, d)),
                  _perb((nb, n, d)), _full((p2p, n)), _full((p2p, n))],
        out_specs=[_perb((nb, n, d)), _full((d, d)), _full((1, d))],
        out_shape=[jax.ShapeDtypeStruct((gb, nb, n, d), _F32),
                   jax.ShapeDtypeStruct((d, d), _F32),
                   jax.ShapeDtypeStruct((1, d), _F32)],
    )(pm.reshape(gb, nb, newn, n), xp.reshape(gb, nb, newn, d),
      down.reshape(gb, nb, n, d), r2, t2)
    return o.reshape(b, n, d), (m, sd)


def kernel(A_init, X, params):
    org_x = X
    a_new = _pallas_mlp(params['start_mlp'], X)
    x, ms = _pallas_gcn_mom(params['start_gcn'], a_new, A_init, X)
    adj, downs, pms = [], [], []
    for i in range(_LN):
        a_old = a_new
        a_new = _pallas_mlp(params['down_mlp_%d' % i], x, ms)
        x = _pallas_gcn(params['down_gcn_%d' % i], a_new, a_old, x)
        adj.append(a_new)
        downs.append(x)
        a_new, x, pm, ms = _pallas_pool(params['pool_%d' % i], _KS[i],
                                        a_new, x)
        pms.append(pm)
    a_old = a_new
    a_new = _pallas_mlp(params['bottom_mlp'], x, ms)
    x = _pallas_gcn(params['bottom_gcn'], a_new, a_old, x)
    for i in range(_LN):
        u = _LN - 1 - i
        a_old = adj[u]
        x, ms = _pallas_unpool_add(pms[u], x, downs[u])
        a_new = _pallas_mlp(params['up_mlp_%d' % u], x, ms)
        if u > 0:
            x = _pallas_gcn(params['up_gcn_%d' % u], a_new, a_old, x)
        else:
            x, ms = _pallas_gcn_mom(params['up_gcn_0'], a_new, a_old, x,
                                    org=org_x)
    x = jnp.concatenate([x, org_x], axis=-1)
    a_old = a_new
    a_new = _pallas_mlp(params['out_mlp'], x, ms)
    return _pallas_gcn(params['out_gcn'], a_new, a_old, x, logsm=True)
